# baseline (speedup 1.0000x reference)
import jax
import jax.numpy as jnp
from jax import lax
from jax.experimental import pallas as pl
from jax.experimental.pallas import tpu as pltpu
from jax.experimental.pallas import tpu_sc as plsc

BATCH = 4096
SEQ = 200
EMBED = 32
VOCAB = 1000000
NC = 2
NS = 16
NW = NC * NS
BPW = BATCH // NW


def _pool_body(xt_hbm, dst_hbm, emb_hbm, out_hbm,
               idx_v, dst_v, buf0, buf1, slab_sh, sem0, sem1):
    c = lax.axis_index("c")
    s = lax.axis_index("s")
    wid = s * NC + c

    pltpu.sync_copy(xt_hbm.at[:, pl.ds(wid * BPW, BPW)], idx_v)
    pltpu.sync_copy(dst_hbm.at[s], dst_v)

    def _zero(i, _):
        z = jnp.zeros((16,), jnp.float32)
        buf0[i, pl.ds(0, 16)] = z
        buf0[i, pl.ds(16, 16)] = z
        return 0
    lax.fori_loop(0, BPW, _zero, 0)
    pltpu.sync_copy(buf0, slab_sh.at[pl.ds(s * BPW, BPW)])

    def _start(chunk, buf, sem):
        return pltpu.async_copy(emb_hbm.at[idx_v.at[chunk]], buf, sem)

    def _wait(chunk, buf, sem):
        pltpu.make_async_copy(emb_hbm.at[idx_v.at[chunk]], buf, sem).wait()

    def _scadd(buf):
        pltpu.sync_copy(buf, slab_sh.at[dst_v], add=True)

    def body(i, _):
        a = 2 * i
        b = 2 * i + 1
        _start(b, buf1, sem1)
        _wait(a, buf0, sem0)
        _scadd(buf0)

        @pl.when(i < SEQ // 2 - 1)
        def _():
            _start(a + 2, buf0, sem0)

        _wait(b, buf1, sem1)
        _scadd(buf1)
        return 0

    _start(0, buf0, sem0)
    lax.fori_loop(0, SEQ // 2, body, 0)

    pltpu.sync_copy(slab_sh.at[pl.ds(s * BPW, BPW)],
                    out_hbm.at[pl.ds(wid * BPW, BPW)])


def _sc_pool(xt, dst, embedding):
    mesh = plsc.VectorSubcoreMesh(core_axis_name="c", subcore_axis_name="s")
    return pl.kernel(
        _pool_body,
        out_type=jax.ShapeDtypeStruct((BATCH, EMBED), jnp.float32),
        mesh=mesh,
        compiler_params=pltpu.CompilerParams(use_tc_tiling_on_sc=False),
        scratch_types=[
            pltpu.VMEM((SEQ, BPW), jnp.int32),
            pltpu.VMEM((BPW,), jnp.int32),
            pltpu.VMEM((BPW, EMBED), jnp.float32),
            pltpu.VMEM((BPW, EMBED), jnp.float32),
            pltpu.VMEM_SHARED((NS * BPW, EMBED), jnp.float32),
            pltpu.SemaphoreType.DMA,
            pltpu.SemaphoreType.DMA,
        ],
    )(xt, dst, embedding)


CW = 128
COLS_W = 31232
NCH_W = COLS_W // CW
VOCAB_PAD = 1000064
NCH_LAST = (VOCAB_PAD - 31 * COLS_W) // CW


def _fmt_body(et_hbm, out_hbm, vin, vout):
    c = lax.axis_index("c")
    s = lax.axis_index("s")
    wid = s * NC + c
    base_col = wid * COLS_W
    iota32 = lax.iota(jnp.int32, 16) * 32

    def do_chunk(col0, ncols16):
        pltpu.sync_copy(et_hbm.at[:, pl.ds(col0, 16 * ncols16)],
                        vin.at[:, pl.ds(0, 16 * ncols16)])

        def grp(g, _):
            for cc in range(EMBED):
                v = vin[cc, pl.ds(g * 16, 16)]
                plsc.store_scatter(vout, [iota32 + (g * 512 + cc)], v)
            return 0
        lax.fori_loop(0, ncols16, grp, 0)
        pltpu.sync_copy(vout.at[pl.ds(0, 16 * ncols16 * EMBED)],
                        out_hbm.at[pl.ds(col0 * EMBED,
                                         16 * ncols16 * EMBED)])

    def body(i, _):
        do_chunk(base_col + i * CW, CW // 16)
        return 0

    nch = jnp.where(wid == NW - 1, NCH_LAST, NCH_W)
    lax.fori_loop(0, nch, body, 0)


def _sc_format(emb_t):
    mesh = plsc.VectorSubcoreMesh(core_axis_name="c", subcore_axis_name="s")
    return pl.kernel(
        _fmt_body,
        out_type=jax.ShapeDtypeStruct((VOCAB_PAD * EMBED,), jnp.float32),
        mesh=mesh,
        compiler_params=pltpu.CompilerParams(use_tc_tiling_on_sc=True,
                                             needs_layout_passes=False,
                                             disable_bounds_checks=True),
        scratch_types=[
            pltpu.VMEM((EMBED, CW), jnp.float32),
            pltpu.VMEM((CW * EMBED,), jnp.float32),
        ],
    )(emb_t)


def _mlp_body(p_ref, w1_ref, b1_ref, w2_ref, b2_ref, o_ref):
    p = p_ref[...] * jnp.float32(1.0 / SEQ)
    h = jnp.dot(p, w1_ref[...], preferred_element_type=jnp.float32,
                precision=lax.Precision.HIGHEST)
    h = jnp.maximum(h + b1_ref[...], 0.0)
    o_ref[...] = jnp.dot(h, w2_ref[...], preferred_element_type=jnp.float32,
                         precision=lax.Precision.HIGHEST) + b2_ref[...]


def _tc_mlp(pooled, fc1_w, fc1_b, fc2_w, fc2_b):
    return pl.pallas_call(
        _mlp_body,
        out_shape=jax.ShapeDtypeStruct((BATCH, fc2_w.shape[1]), jnp.float32),
    )(pooled, fc1_w, fc1_b.reshape(1, -1), fc2_w, fc2_b.reshape(1, -1))


def kernel(x, embedding, fc1_w, fc1_b, fc2_w, fc2_b):
    xt = x.T
    dst = (jnp.arange(NS, dtype=jnp.int32)[:, None] * BPW
           + jnp.arange(BPW, dtype=jnp.int32)[None, :])
    emb_lin = _sc_format(embedding.T).reshape(VOCAB_PAD, EMBED)
    sums = _sc_pool(xt, dst, emb_lin)
    return _tc_mlp(sums, fc1_w, fc1_b, fc2_w, fc2_b)


# speedup vs baseline: 1.3271x; 1.3271x over previous
import jax
import jax.numpy as jnp
from jax import lax
from jax.experimental import pallas as pl
from jax.experimental.pallas import tpu as pltpu
from jax.experimental.pallas import tpu_sc as plsc

BATCH = 4096
SEQ = 200
EMBED = 32
VOCAB = 1000000
NC = 2
NS = 16
NW = NC * NS
BPW = BATCH // NW


def _pool_body(xt_hbm, dst_hbm, emb_hbm, out_hbm,
               idx_v, dst_v, buf0, buf1, slab_sh, sem0, sem1):
    c = lax.axis_index("c")
    s = lax.axis_index("s")
    wid = s * NC + c

    pltpu.sync_copy(xt_hbm.at[:, pl.ds(wid * BPW, BPW)], idx_v)
    pltpu.sync_copy(dst_hbm.at[s], dst_v)

    def _zero(i, _):
        z = jnp.zeros((16,), jnp.float32)
        buf0[i, pl.ds(0, 16)] = z
        buf0[i, pl.ds(16, 16)] = z
        return 0
    lax.fori_loop(0, BPW, _zero, 0)
    pltpu.sync_copy(buf0, slab_sh.at[pl.ds(s * BPW, BPW)])

    def _start(chunk, buf, sem):
        return pltpu.async_copy(emb_hbm.at[idx_v.at[chunk]], buf, sem)

    def _wait(chunk, buf, sem):
        pltpu.make_async_copy(emb_hbm.at[idx_v.at[chunk]], buf, sem).wait()

    def _scadd(buf):
        pltpu.sync_copy(buf, slab_sh.at[dst_v], add=True)

    def body(i, _):
        a = 2 * i
        b = 2 * i + 1
        _start(b, buf1, sem1)
        _wait(a, buf0, sem0)
        _scadd(buf0)

        @pl.when(i < SEQ // 2 - 1)
        def _():
            _start(a + 2, buf0, sem0)

        _wait(b, buf1, sem1)
        _scadd(buf1)
        return 0

    _start(0, buf0, sem0)
    lax.fori_loop(0, SEQ // 2, body, 0)

    pltpu.sync_copy(slab_sh.at[pl.ds(s * BPW, BPW)],
                    out_hbm.at[pl.ds(wid * BPW, BPW)])


def _sc_pool(xt, dst, embedding):
    mesh = plsc.VectorSubcoreMesh(core_axis_name="c", subcore_axis_name="s")
    return pl.kernel(
        _pool_body,
        out_type=jax.ShapeDtypeStruct((BATCH, EMBED), jnp.float32),
        mesh=mesh,
        compiler_params=pltpu.CompilerParams(use_tc_tiling_on_sc=False),
        scratch_types=[
            pltpu.VMEM((SEQ, BPW), jnp.int32),
            pltpu.VMEM((BPW,), jnp.int32),
            pltpu.VMEM((BPW, EMBED), jnp.float32),
            pltpu.VMEM((BPW, EMBED), jnp.float32),
            pltpu.VMEM_SHARED((NS * BPW, EMBED), jnp.float32),
            pltpu.SemaphoreType.DMA,
            pltpu.SemaphoreType.DMA,
        ],
    )(xt, dst, embedding)


CW = 128
COLS_W = 31232
NCH_W = COLS_W // CW
VOCAB_PAD = 1000064
NCH_LAST = (VOCAB_PAD - 31 * COLS_W) // CW


def _fmt_body(et_hbm, out_hbm, vin0, vin1, vout0, vout1,
              gsem0, gsem1, osem0, osem1):
    c = lax.axis_index("c")
    s = lax.axis_index("s")
    wid = s * NC + c
    base_col = wid * COLS_W
    iota32 = lax.iota(jnp.int32, 16) * 32
    nch = jnp.where(wid == NW - 1, NCH_LAST, NCH_W)

    def gstart(i, vin, gsem):
        pltpu.async_copy(et_hbm.at[:, pl.ds(base_col + i * CW, CW)],
                         vin, gsem)

    def owait(i, vout, osem):
        pltpu.make_async_copy(
            vout, out_hbm.at[pl.ds((base_col + i * CW) * EMBED, CW * EMBED)],
            osem).wait()

    def process(i, vin, vout, gsem, osem):
        pltpu.make_async_copy(et_hbm.at[:, pl.ds(base_col + i * CW, CW)],
                              vin, gsem).wait()

        @pl.when(i >= 2)
        def _():
            owait(i - 2, vout, osem)

        def grp(g, _):
            for cc in range(EMBED):
                v = vin[cc, pl.ds(g * 16, 16)]
                plsc.store_scatter(vout, [iota32 + (g * 512 + cc)], v)
            return 0
        lax.fori_loop(0, CW // 16, grp, 0)
        pltpu.async_copy(
            vout, out_hbm.at[pl.ds((base_col + i * CW) * EMBED, CW * EMBED)],
            osem)

        @pl.when(i + 2 < nch)
        def _():
            gstart(i + 2, vin, gsem)

    def body(i, _):
        @pl.when(i % 2 == 0)
        def _():
            process(i, vin0, vout0, gsem0, osem0)

        @pl.when(i % 2 == 1)
        def _():
            process(i, vin1, vout1, gsem1, osem1)
        return 0

    gstart(0, vin0, gsem0)
    gstart(1, vin1, gsem1)
    lax.fori_loop(0, nch, body, 0)

    @pl.when(wid < NW - 1)
    def _():
        owait(NCH_W - 2, vout0, osem0)
        owait(NCH_W - 1, vout1, osem1)

    @pl.when(wid == NW - 1)
    def _():
        owait(NCH_LAST - 1, vout0, osem0)
        owait(NCH_LAST - 2, vout1, osem1)


def _sc_format(emb_t):
    mesh = plsc.VectorSubcoreMesh(core_axis_name="c", subcore_axis_name="s")
    return pl.kernel(
        _fmt_body,
        out_type=jax.ShapeDtypeStruct((VOCAB_PAD * EMBED,), jnp.float32),
        mesh=mesh,
        compiler_params=pltpu.CompilerParams(use_tc_tiling_on_sc=True,
                                             needs_layout_passes=False,
                                             disable_bounds_checks=True),
        scratch_types=[
            pltpu.VMEM((EMBED, CW), jnp.float32),
            pltpu.VMEM((EMBED, CW), jnp.float32),
            pltpu.VMEM((CW * EMBED,), jnp.float32),
            pltpu.VMEM((CW * EMBED,), jnp.float32),
            pltpu.SemaphoreType.DMA,
            pltpu.SemaphoreType.DMA,
            pltpu.SemaphoreType.DMA,
            pltpu.SemaphoreType.DMA,
        ],
    )(emb_t)


def _mlp_body(p_ref, w1_ref, b1_ref, w2_ref, b2_ref, o_ref):
    p = p_ref[...] * jnp.float32(1.0 / SEQ)
    h = jnp.dot(p, w1_ref[...], preferred_element_type=jnp.float32,
                precision=lax.Precision.HIGHEST)
    h = jnp.maximum(h + b1_ref[...], 0.0)
    o_ref[...] = jnp.dot(h, w2_ref[...], preferred_element_type=jnp.float32,
                         precision=lax.Precision.HIGHEST) + b2_ref[...]


def _tc_mlp(pooled, fc1_w, fc1_b, fc2_w, fc2_b):
    return pl.pallas_call(
        _mlp_body,
        out_shape=jax.ShapeDtypeStruct((BATCH, fc2_w.shape[1]), jnp.float32),
    )(pooled, fc1_w, fc1_b.reshape(1, -1), fc2_w, fc2_b.reshape(1, -1))


def kernel(x, embedding, fc1_w, fc1_b, fc2_w, fc2_b):
    xt = x.T
    dst = (jnp.arange(NS, dtype=jnp.int32)[:, None] * BPW
           + jnp.arange(BPW, dtype=jnp.int32)[None, :])
    emb_lin = _sc_format(embedding.T).reshape(VOCAB_PAD, EMBED)
    sums = _sc_pool(xt, dst, emb_lin)
    return _tc_mlp(sums, fc1_w, fc1_b, fc2_w, fc2_b)


# speedup vs baseline: 2.3579x; 1.7768x over previous
import jax
import jax.numpy as jnp
from jax import lax
from jax.experimental import pallas as pl
from jax.experimental.pallas import tpu as pltpu
from jax.experimental.pallas import tpu_sc as plsc

BATCH = 4096
SEQ = 200
EMBED = 32
VOCAB = 1000000
NC = 2
NS = 16
NW = NC * NS
BPW = BATCH // NW


def _pool_body(xt_hbm, dst_hbm, emb_hbm, out_hbm,
               idx_v, dst_v, buf0, buf1, slab_sh, sem0, sem1):
    c = lax.axis_index("c")
    s = lax.axis_index("s")
    wid = s * NC + c

    pltpu.sync_copy(xt_hbm.at[:, pl.ds(wid * BPW, BPW)], idx_v)
    pltpu.sync_copy(dst_hbm.at[s], dst_v)

    def _zero(i, _):
        z = jnp.zeros((16,), jnp.float32)
        buf0[i, pl.ds(0, 16)] = z
        buf0[i, pl.ds(16, 16)] = z
        return 0
    lax.fori_loop(0, BPW, _zero, 0)
    pltpu.sync_copy(buf0, slab_sh.at[pl.ds(s * BPW, BPW)])

    def _start(chunk, buf, sem):
        return pltpu.async_copy(emb_hbm.at[idx_v.at[chunk]], buf, sem)

    def _wait(chunk, buf, sem):
        pltpu.make_async_copy(emb_hbm.at[idx_v.at[chunk]], buf, sem).wait()

    def _scadd(buf):
        pltpu.sync_copy(buf, slab_sh.at[dst_v], add=True)

    def body(i, _):
        a = 2 * i
        b = 2 * i + 1
        _start(b, buf1, sem1)
        _wait(a, buf0, sem0)
        _scadd(buf0)

        @pl.when(i < SEQ // 2 - 1)
        def _():
            _start(a + 2, buf0, sem0)

        _wait(b, buf1, sem1)
        _scadd(buf1)
        return 0

    _start(0, buf0, sem0)
    lax.fori_loop(0, SEQ // 2, body, 0)

    pltpu.sync_copy(slab_sh.at[pl.ds(s * BPW, BPW)],
                    out_hbm.at[pl.ds(wid * BPW, BPW)])


def _sc_pool(xt, dst, embedding):
    mesh = plsc.VectorSubcoreMesh(core_axis_name="c", subcore_axis_name="s")
    return pl.kernel(
        _pool_body,
        out_type=jax.ShapeDtypeStruct((BATCH, EMBED), jnp.float32),
        mesh=mesh,
        compiler_params=pltpu.CompilerParams(use_tc_tiling_on_sc=False),
        scratch_types=[
            pltpu.VMEM((SEQ, BPW), jnp.int32),
            pltpu.VMEM((BPW,), jnp.int32),
            pltpu.VMEM((BPW, EMBED), jnp.float32),
            pltpu.VMEM((BPW, EMBED), jnp.float32),
            pltpu.VMEM_SHARED((NS * BPW, EMBED), jnp.float32),
            pltpu.SemaphoreType.DMA,
            pltpu.SemaphoreType.DMA,
        ],
    )(xt, dst, embedding)


CW = 128
COLS_W = 31232
NCH_W = COLS_W // CW
VOCAB_PAD = 1000064
NCH_LAST = (VOCAB_PAD - 31 * COLS_W) // CW


def _fmt_body(et_hbm, out_hbm, vin0, vin1, vout0, vout1,
              gsem0, gsem1, osem0, osem1):
    c = lax.axis_index("c")
    s = lax.axis_index("s")
    wid = s * NC + c
    base_col = wid * COLS_W
    iota = lax.iota(jnp.int32, 16)
    iota32 = iota * 32
    nch = jnp.where(wid == NW - 1, NCH_LAST, NCH_W)

    def gstart(i, vin, gsem):
        pltpu.async_copy(et_hbm.at[:, pl.ds(base_col + i * CW, CW)],
                         vin, gsem)

    def owait(i, vout, osem):
        pltpu.make_async_copy(
            vout, out_hbm.at[pl.ds((base_col + i * CW) * EMBED, CW * EMBED)],
            osem).wait()

    def process(i, vin, vout, gsem, osem):
        pltpu.make_async_copy(et_hbm.at[:, pl.ds(base_col + i * CW, CW)],
                              vin, gsem).wait()

        @pl.when(i >= 2)
        def _():
            owait(i - 2, vout, osem)

        def grp(g, _):
            colv = iota + g * 16
            for d in range(16):
                r = iota + d
                r = jnp.where(r >= 16, r - 16, r)
                sbase = iota32 + r + g * 512
                for h in (0, 1):
                    v = plsc.load_gather(vin, [r + 16 * h, colv])
                    plsc.store_scatter(vout, [sbase + 16 * h], v)
            return 0
        lax.fori_loop(0, CW // 16, grp, 0)
        pltpu.async_copy(
            vout, out_hbm.at[pl.ds((base_col + i * CW) * EMBED, CW * EMBED)],
            osem)

        @pl.when(i + 2 < nch)
        def _():
            gstart(i + 2, vin, gsem)

    def body(i, _):
        @pl.when(i % 2 == 0)
        def _():
            process(i, vin0, vout0, gsem0, osem0)

        @pl.when(i % 2 == 1)
        def _():
            process(i, vin1, vout1, gsem1, osem1)
        return 0

    gstart(0, vin0, gsem0)
    gstart(1, vin1, gsem1)
    lax.fori_loop(0, nch, body, 0)

    @pl.when(wid < NW - 1)
    def _():
        owait(NCH_W - 2, vout0, osem0)
        owait(NCH_W - 1, vout1, osem1)

    @pl.when(wid == NW - 1)
    def _():
        owait(NCH_LAST - 1, vout0, osem0)
        owait(NCH_LAST - 2, vout1, osem1)


def _sc_format(emb_t):
    mesh = plsc.VectorSubcoreMesh(core_axis_name="c", subcore_axis_name="s")
    return pl.kernel(
        _fmt_body,
        out_type=jax.ShapeDtypeStruct((VOCAB_PAD * EMBED,), jnp.float32),
        mesh=mesh,
        compiler_params=pltpu.CompilerParams(use_tc_tiling_on_sc=True,
                                             needs_layout_passes=False,
                                             disable_bounds_checks=True),
        scratch_types=[
            pltpu.VMEM((EMBED, CW), jnp.float32),
            pltpu.VMEM((EMBED, CW), jnp.float32),
            pltpu.VMEM((CW * EMBED,), jnp.float32),
            pltpu.VMEM((CW * EMBED,), jnp.float32),
            pltpu.SemaphoreType.DMA,
            pltpu.SemaphoreType.DMA,
            pltpu.SemaphoreType.DMA,
            pltpu.SemaphoreType.DMA,
        ],
    )(emb_t)


def _mlp_body(p_ref, w1_ref, b1_ref, w2_ref, b2_ref, o_ref):
    p = p_ref[...] * jnp.float32(1.0 / SEQ)
    h = jnp.dot(p, w1_ref[...], preferred_element_type=jnp.float32,
                precision=lax.Precision.HIGHEST)
    h = jnp.maximum(h + b1_ref[...], 0.0)
    o_ref[...] = jnp.dot(h, w2_ref[...], preferred_element_type=jnp.float32,
                         precision=lax.Precision.HIGHEST) + b2_ref[...]


def _tc_mlp(pooled, fc1_w, fc1_b, fc2_w, fc2_b):
    return pl.pallas_call(
        _mlp_body,
        out_shape=jax.ShapeDtypeStruct((BATCH, fc2_w.shape[1]), jnp.float32),
    )(pooled, fc1_w, fc1_b.reshape(1, -1), fc2_w, fc2_b.reshape(1, -1))


def kernel(x, embedding, fc1_w, fc1_b, fc2_w, fc2_b):
    xt = x.T
    dst = (jnp.arange(NS, dtype=jnp.int32)[:, None] * BPW
           + jnp.arange(BPW, dtype=jnp.int32)[None, :])
    emb_lin = _sc_format(embedding.T).reshape(VOCAB_PAD, EMBED)
    sums = _sc_pool(xt, dst, emb_lin)
    return _tc_mlp(sums, fc1_w, fc1_b, fc2_w, fc2_b)


# speedup vs baseline: 2.3976x; 1.0168x over previous
import jax
import jax.numpy as jnp
from jax import lax
from jax.experimental import pallas as pl
from jax.experimental.pallas import tpu as pltpu
from jax.experimental.pallas import tpu_sc as plsc

BATCH = 4096
SEQ = 200
EMBED = 32
VOCAB = 1000000
NC = 2
NS = 16
NW = NC * NS
BPW = BATCH // NW


def _pool_body(xt_hbm, dst_hbm, emb_hbm, out_hbm,
               idx_v, dst_v, buf0, buf1, slab_sh, sem0, sem1):
    c = lax.axis_index("c")
    s = lax.axis_index("s")
    wid = s * NC + c

    pltpu.sync_copy(xt_hbm.at[:, pl.ds(wid * BPW, BPW)], idx_v)
    pltpu.sync_copy(dst_hbm.at[s], dst_v)

    def _zero(i, _):
        z = jnp.zeros((16,), jnp.float32)
        buf0[i, pl.ds(0, 16)] = z
        buf0[i, pl.ds(16, 16)] = z
        return 0
    lax.fori_loop(0, BPW, _zero, 0)
    pltpu.sync_copy(buf0, slab_sh.at[pl.ds(s * BPW, BPW)])

    def _start(chunk, buf, sem):
        return pltpu.async_copy(emb_hbm.at[idx_v.at[chunk]], buf, sem)

    def _wait(chunk, buf, sem):
        pltpu.make_async_copy(emb_hbm.at[idx_v.at[chunk]], buf, sem).wait()

    def _scadd(buf):
        pltpu.sync_copy(buf, slab_sh.at[dst_v], add=True)

    def body(i, _):
        a = 2 * i
        b = 2 * i + 1
        _start(b, buf1, sem1)
        _wait(a, buf0, sem0)
        _scadd(buf0)

        @pl.when(i < SEQ // 2 - 1)
        def _():
            _start(a + 2, buf0, sem0)

        _wait(b, buf1, sem1)
        _scadd(buf1)
        return 0

    _start(0, buf0, sem0)
    lax.fori_loop(0, SEQ // 2, body, 0)

    pltpu.sync_copy(slab_sh.at[pl.ds(s * BPW, BPW)],
                    out_hbm.at[pl.ds(wid * BPW, BPW)])


def _sc_pool(xt, dst, embedding):
    mesh = plsc.VectorSubcoreMesh(core_axis_name="c", subcore_axis_name="s")
    return pl.kernel(
        _pool_body,
        out_type=jax.ShapeDtypeStruct((BATCH, EMBED), jnp.float32),
        mesh=mesh,
        compiler_params=pltpu.CompilerParams(use_tc_tiling_on_sc=False),
        scratch_types=[
            pltpu.VMEM((SEQ, BPW), jnp.int32),
            pltpu.VMEM((BPW,), jnp.int32),
            pltpu.VMEM((BPW, EMBED), jnp.float32),
            pltpu.VMEM((BPW, EMBED), jnp.float32),
            pltpu.VMEM_SHARED((NS * BPW, EMBED), jnp.float32),
            pltpu.SemaphoreType.DMA,
            pltpu.SemaphoreType.DMA,
        ],
    )(xt, dst, embedding)


CW = 256
COLS_W = 31232
NCH_W = COLS_W // CW
VOCAB_PAD = 1000064
NCH_LAST = (VOCAB_PAD - 31 * COLS_W) // CW
TAILC = VOCAB_PAD - 31 * COLS_W - NCH_LAST * CW


def _fmt_body(et_hbm, out_hbm, vin0, vin1, vout0, vout1,
              gsem0, gsem1, osem0, osem1):
    c = lax.axis_index("c")
    s = lax.axis_index("s")
    wid = s * NC + c
    base_col = wid * COLS_W
    iota = lax.iota(jnp.int32, 16)
    iota32 = iota * 32
    nch = jnp.where(wid == NW - 1, NCH_LAST, NCH_W)

    def gstart(i, vin, gsem):
        pltpu.async_copy(et_hbm.at[:, pl.ds(base_col + i * CW, CW)],
                         vin, gsem)

    def owait(i, vout, osem):
        pltpu.make_async_copy(
            vout, out_hbm.at[pl.ds((base_col + i * CW) * EMBED, CW * EMBED)],
            osem).wait()

    def transpose_chunk(vin, vout, ngroups):
        def grp(g, _):
            colv = iota + g * 16
            for d in range(16):
                r = iota + d
                r = jnp.where(r >= 16, r - 16, r)
                sbase = iota32 + r + g * 512
                for h in (0, 1):
                    v = plsc.load_gather(vin, [r + 16 * h, colv])
                    plsc.store_scatter(vout, [sbase + 16 * h], v)
            return 0
        lax.fori_loop(0, ngroups, grp, 0)

    def process(i, vin, vout, gsem, osem):
        pltpu.make_async_copy(et_hbm.at[:, pl.ds(base_col + i * CW, CW)],
                              vin, gsem).wait()

        @pl.when(i >= 2)
        def _():
            owait(i - 2, vout, osem)

        transpose_chunk(vin, vout, CW // 16)
        pltpu.async_copy(
            vout, out_hbm.at[pl.ds((base_col + i * CW) * EMBED, CW * EMBED)],
            osem)

        @pl.when(i + 2 < nch)
        def _():
            gstart(i + 2, vin, gsem)

    def body(i, _):
        @pl.when(i % 2 == 0)
        def _():
            process(i, vin0, vout0, gsem0, osem0)

        @pl.when(i % 2 == 1)
        def _():
            process(i, vin1, vout1, gsem1, osem1)
        return 0

    gstart(0, vin0, gsem0)
    gstart(1, vin1, gsem1)
    lax.fori_loop(0, nch, body, 0)

    @pl.when(wid < NW - 1)
    def _():
        owait(NCH_W - 2, vout0, osem0)
        owait(NCH_W - 1, vout1, osem1)

    @pl.when(wid == NW - 1)
    def _():
        owait(NCH_LAST - 2, vout0, osem0)
        owait(NCH_LAST - 1, vout1, osem1)
        tcol = wid * COLS_W + NCH_LAST * CW
        pltpu.sync_copy(et_hbm.at[:, pl.ds(tcol, TAILC)],
                        vin0.at[:, pl.ds(0, TAILC)])
        transpose_chunk(vin0, vout0, TAILC // 16)
        pltpu.sync_copy(vout0.at[pl.ds(0, TAILC * EMBED)],
                        out_hbm.at[pl.ds(tcol * EMBED, TAILC * EMBED)])


def _sc_format(emb_t):
    mesh = plsc.VectorSubcoreMesh(core_axis_name="c", subcore_axis_name="s")
    return pl.kernel(
        _fmt_body,
        out_type=jax.ShapeDtypeStruct((VOCAB_PAD * EMBED,), jnp.float32),
        mesh=mesh,
        compiler_params=pltpu.CompilerParams(use_tc_tiling_on_sc=True,
                                             needs_layout_passes=False,
                                             disable_bounds_checks=True),
        scratch_types=[
            pltpu.VMEM((EMBED, CW), jnp.float32),
            pltpu.VMEM((EMBED, CW), jnp.float32),
            pltpu.VMEM((CW * EMBED,), jnp.float32),
            pltpu.VMEM((CW * EMBED,), jnp.float32),
            pltpu.SemaphoreType.DMA,
            pltpu.SemaphoreType.DMA,
            pltpu.SemaphoreType.DMA,
            pltpu.SemaphoreType.DMA,
        ],
    )(emb_t)


def _mlp_body(p_ref, w1_ref, b1_ref, w2_ref, b2_ref, o_ref):
    p = p_ref[...] * jnp.float32(1.0 / SEQ)
    h = jnp.dot(p, w1_ref[...], preferred_element_type=jnp.float32,
                precision=lax.Precision.HIGHEST)
    h = jnp.maximum(h + b1_ref[...], 0.0)
    o_ref[...] = jnp.dot(h, w2_ref[...], preferred_element_type=jnp.float32,
                         precision=lax.Precision.HIGHEST) + b2_ref[...]


def _tc_mlp(pooled, fc1_w, fc1_b, fc2_w, fc2_b):
    return pl.pallas_call(
        _mlp_body,
        out_shape=jax.ShapeDtypeStruct((BATCH, fc2_w.shape[1]), jnp.float32),
    )(pooled, fc1_w, fc1_b.reshape(1, -1), fc2_w, fc2_b.reshape(1, -1))


def kernel(x, embedding, fc1_w, fc1_b, fc2_w, fc2_b):
    xt = x.T
    dst = (jnp.arange(NS, dtype=jnp.int32)[:, None] * BPW
           + jnp.arange(BPW, dtype=jnp.int32)[None, :])
    emb_lin = _sc_format(embedding.T).reshape(VOCAB_PAD, EMBED)
    sums = _sc_pool(xt, dst, emb_lin)
    return _tc_mlp(sums, fc1_w, fc1_b, fc2_w, fc2_b)


# speedup vs baseline: 2.6383x; 1.1004x over previous
import jax
import jax.numpy as jnp
from jax import lax
from jax.experimental import pallas as pl
from jax.experimental.pallas import tpu as pltpu
from jax.experimental.pallas import tpu_sc as plsc

BATCH = 4096
SEQ = 200
EMBED = 32
VOCAB = 1000000
NC = 2
NS = 16
NW = NC * NS
BPW = BATCH // NW


def _pool_body(xt_hbm, dst_hbm, emb_hbm, out_hbm,
               idx_v, dst_v, buf0, buf1, slab_sh, sem0, sem1):
    c = lax.axis_index("c")
    s = lax.axis_index("s")
    wid = s * NC + c

    pltpu.sync_copy(xt_hbm.at[:, pl.ds(wid * BPW, BPW)], idx_v)
    pltpu.sync_copy(dst_hbm.at[s], dst_v)

    def _zero(i, _):
        z = jnp.zeros((16,), jnp.float32)
        buf0[i, pl.ds(0, 16)] = z
        buf0[i, pl.ds(16, 16)] = z
        return 0
    lax.fori_loop(0, BPW, _zero, 0)
    pltpu.sync_copy(buf0, slab_sh.at[pl.ds(s * BPW, BPW)])

    def _start(chunk, buf, sem):
        return pltpu.async_copy(emb_hbm.at[idx_v.at[chunk]], buf, sem)

    def _wait(chunk, buf, sem):
        pltpu.make_async_copy(emb_hbm.at[idx_v.at[chunk]], buf, sem).wait()

    def _scadd(buf):
        pltpu.sync_copy(buf, slab_sh.at[dst_v], add=True)

    def body(i, _):
        a = 2 * i
        b = 2 * i + 1
        _start(b, buf1, sem1)
        _wait(a, buf0, sem0)
        _scadd(buf0)

        @pl.when(i < SEQ // 2 - 1)
        def _():
            _start(a + 2, buf0, sem0)

        _wait(b, buf1, sem1)
        _scadd(buf1)
        return 0

    _start(0, buf0, sem0)
    lax.fori_loop(0, SEQ // 2, body, 0)

    pltpu.sync_copy(slab_sh.at[pl.ds(s * BPW, BPW)],
                    out_hbm.at[pl.ds(wid * BPW, BPW)])


def _sc_pool(xt, dst, embedding):
    mesh = plsc.VectorSubcoreMesh(core_axis_name="c", subcore_axis_name="s")
    return pl.kernel(
        _pool_body,
        out_type=jax.ShapeDtypeStruct((BATCH, EMBED), jnp.float32),
        mesh=mesh,
        compiler_params=pltpu.CompilerParams(use_tc_tiling_on_sc=False),
        scratch_types=[
            pltpu.VMEM((SEQ, BPW), jnp.int32),
            pltpu.VMEM((BPW,), jnp.int32),
            pltpu.VMEM((BPW, EMBED), jnp.float32),
            pltpu.VMEM((BPW, EMBED), jnp.float32),
            pltpu.VMEM_SHARED((NS * BPW, EMBED), jnp.float32),
            pltpu.SemaphoreType.DMA,
            pltpu.SemaphoreType.DMA,
        ],
    )(xt, dst, embedding)


CW = 256
COLS_W = 31232
NCH_W = COLS_W // CW
VOCAB_PAD = 1000064
NCH_LAST = (VOCAB_PAD - 31 * COLS_W) // CW
TAILC = VOCAB_PAD - 31 * COLS_W - NCH_LAST * CW


def _fmt_body(et_hbm, out_hbm, vin0, vin1, vout0, vout1,
              gsem0, gsem1, osem0, osem1):
    c = lax.axis_index("c")
    s = lax.axis_index("s")
    wid = s * NC + c
    base_col = wid * COLS_W
    iota = lax.iota(jnp.int32, 16)
    iota32 = iota * 32
    nch = jnp.where(wid == NW - 1, NCH_LAST, NCH_W)

    def gstart(i, vin, gsem):
        pltpu.async_copy(et_hbm.at[:, pl.ds(base_col + i * CW, CW)],
                         vin, gsem)

    def owait(i, vout, osem):
        pltpu.make_async_copy(
            vout, out_hbm.at[pl.ds((base_col + i * CW) * EMBED, CW * EMBED)],
            osem).wait()

    def transpose_chunk(vin, vout, ngroups):
        @plsc.parallel_loop(0, ngroups, unroll=2)
        def grp(g):
            colv = iota + g * 16
            for d in range(16):
                r = iota + d
                r = jnp.where(r >= 16, r - 16, r)
                sbase = iota32 + r + g * 512
                for h in (0, 1):
                    v = plsc.load_gather(vin, [r + 16 * h, colv])
                    plsc.store_scatter(vout, [sbase + 16 * h], v)

    def process(i, vin, vout, gsem, osem):
        pltpu.make_async_copy(et_hbm.at[:, pl.ds(base_col + i * CW, CW)],
                              vin, gsem).wait()

        @pl.when(i >= 2)
        def _():
            owait(i - 2, vout, osem)

        transpose_chunk(vin, vout, CW // 16)
        pltpu.async_copy(
            vout, out_hbm.at[pl.ds((base_col + i * CW) * EMBED, CW * EMBED)],
            osem)

        @pl.when(i + 2 < nch)
        def _():
            gstart(i + 2, vin, gsem)

    def body(i, _):
        @pl.when(i % 2 == 0)
        def _():
            process(i, vin0, vout0, gsem0, osem0)

        @pl.when(i % 2 == 1)
        def _():
            process(i, vin1, vout1, gsem1, osem1)
        return 0

    gstart(0, vin0, gsem0)
    gstart(1, vin1, gsem1)
    lax.fori_loop(0, nch, body, 0)

    @pl.when(wid < NW - 1)
    def _():
        owait(NCH_W - 2, vout0, osem0)
        owait(NCH_W - 1, vout1, osem1)

    @pl.when(wid == NW - 1)
    def _():
        owait(NCH_LAST - 2, vout0, osem0)
        owait(NCH_LAST - 1, vout1, osem1)
        tcol = wid * COLS_W + NCH_LAST * CW
        pltpu.sync_copy(et_hbm.at[:, pl.ds(tcol, TAILC)],
                        vin0.at[:, pl.ds(0, TAILC)])
        transpose_chunk(vin0, vout0, TAILC // 16)
        pltpu.sync_copy(vout0.at[pl.ds(0, TAILC * EMBED)],
                        out_hbm.at[pl.ds(tcol * EMBED, TAILC * EMBED)])


def _sc_format(emb_t):
    mesh = plsc.VectorSubcoreMesh(core_axis_name="c", subcore_axis_name="s")
    return pl.kernel(
        _fmt_body,
        out_type=jax.ShapeDtypeStruct((VOCAB_PAD * EMBED,), jnp.float32),
        mesh=mesh,
        compiler_params=pltpu.CompilerParams(use_tc_tiling_on_sc=True,
                                             needs_layout_passes=False,
                                             disable_bounds_checks=True),
        scratch_types=[
            pltpu.VMEM((EMBED, CW), jnp.float32),
            pltpu.VMEM((EMBED, CW), jnp.float32),
            pltpu.VMEM((CW * EMBED,), jnp.float32),
            pltpu.VMEM((CW * EMBED,), jnp.float32),
            pltpu.SemaphoreType.DMA,
            pltpu.SemaphoreType.DMA,
            pltpu.SemaphoreType.DMA,
            pltpu.SemaphoreType.DMA,
        ],
    )(emb_t)


def _mlp_body(p_ref, w1_ref, b1_ref, w2_ref, b2_ref, o_ref):
    p = p_ref[...] * jnp.float32(1.0 / SEQ)
    h = jnp.dot(p, w1_ref[...], preferred_element_type=jnp.float32,
                precision=lax.Precision.HIGHEST)
    h = jnp.maximum(h + b1_ref[...], 0.0)
    o_ref[...] = jnp.dot(h, w2_ref[...], preferred_element_type=jnp.float32,
                         precision=lax.Precision.HIGHEST) + b2_ref[...]


def _tc_mlp(pooled, fc1_w, fc1_b, fc2_w, fc2_b):
    return pl.pallas_call(
        _mlp_body,
        out_shape=jax.ShapeDtypeStruct((BATCH, fc2_w.shape[1]), jnp.float32),
    )(pooled, fc1_w, fc1_b.reshape(1, -1), fc2_w, fc2_b.reshape(1, -1))


def kernel(x, embedding, fc1_w, fc1_b, fc2_w, fc2_b):
    xt = x.T
    dst = (jnp.arange(NS, dtype=jnp.int32)[:, None] * BPW
           + jnp.arange(BPW, dtype=jnp.int32)[None, :])
    emb_lin = _sc_format(embedding.T).reshape(VOCAB_PAD, EMBED)
    sums = _sc_pool(xt, dst, emb_lin)
    return _tc_mlp(sums, fc1_w, fc1_b, fc2_w, fc2_b)
